# baseline (speedup 1.0000x reference)
import jax
import jax.numpy as jnp
from jax import lax
from jax.experimental import pallas as pl
from jax.experimental.pallas import tpu as pltpu
from jax.experimental.pallas import tpu_sc as plsc

B = 4096
NS = 26
VOCAB = 100000
D = 16
L = 50
ND = 13

NC = 2
NSUB = 16
NW = NC * NSUB
BW = B // NW
SP_ROWS = BW * NS
CH = 128
SP_CHUNKS = SP_ROWS // CH
HALF_CHUNKS = SP_CHUNKS // 2
STG_ROWS = HALF_CHUNKS * CH * D // 128
SQ_HALF_B = BW // 2
SQ_HALF_ROWS = SQ_HALF_B * L
SQ_HALF_CHUNKS = SQ_HALF_ROWS // CH
LPAD = 64
TAB_ROWS = NS * VOCAB * D // 128


def _sc_body(tab_hbm, stab_hbm, enc_hbm, sqi_hbm, sqp_hbm,
             out_sp_hbm, out_pool_hbm,
             enc_v, sqi_v, sqp_v, seq_v, super_v, stg_v, pool_v, t0_v,
             idma_v, sem, sem2):
    c = lax.axis_index("c")
    s = lax.axis_index("s")
    wid = s * NC + c

    pltpu.sync_copy(enc_hbm.at[pl.ds(wid * SP_CHUNKS, SP_CHUNKS)], enc_v)
    pltpu.sync_copy(sqp_hbm.at[pl.ds(wid * (BW // 2), BW // 2)], sqp_v)
    pltpu.sync_copy(stab_hbm.at[pl.ds(0, 1)], t0_v)
    t0 = t0_v[0, :]

    pltpu.sync_copy(sqi_hbm.at[pl.ds(wid * 2 * SQ_HALF_CHUNKS,
                                     SQ_HALF_CHUNKS)], sqi_v)
    seq_descs = []
    for j in range(SQ_HALF_CHUNKS):
        seq_descs.append(pltpu.async_copy(
            stab_hbm.at[sqi_v.at[j]],
            seq_v.at[pl.ds(j * CH, CH)], sem2))

    for h in range(2):
        def sp_chunk(j, _):
            jj = h * HALF_CHUNKS + j
            for grp in range(8):
                e = enc_v[jj, pl.ds(grp * D, D)]
                idma_v[0, pl.ds(grp * D, D)] = e >> 3
            pltpu.sync_copy(tab_hbm.at[idma_v.at[0]], super_v)
            for grp in range(8):
                e = enc_v[jj, pl.ds(grp * D, D)]
                sub = (e & 7) << 4
                for k in range(D):
                    g = grp * D + k
                    vals = super_v[g, pl.ds(sub[k], D)]
                    stg_v[j * D + (g >> 3), pl.ds((g & 7) << 4, D)] = vals
            return 0

        lax.fori_loop(0, HALF_CHUNKS, sp_chunk, 0)
        pltpu.sync_copy(
            stg_v, out_sp_hbm.at[pl.ds(wid * 2 * STG_ROWS + h * STG_ROWS,
                                       STG_ROWS)])

    for h in range(2):
        for d_ in seq_descs:
            d_.wait()

        def pool_one(b, _):
            acc = seq_v[b * L, :]
            for l in range(1, L):
                acc = acc + seq_v[b * L + l, :]
            gb = h * SQ_HALF_B + b
            nvec = jnp.zeros((D,), jnp.int32)
            for ch in range(LPAD // D):
                ids = sqp_v[gb // 2, pl.ds((gb % 2) * LPAD + ch * D, D)]
                nvec = nvec + jnp.where(ids > 0, 1, 0).astype(jnp.int32)
            n = nvec[0]
            for i in range(1, D):
                n = n + nvec[i]
            nb = lax.broadcast_in_dim(n.astype(jnp.float32), (D,), ())
            pooled = (acc - (50.0 - nb) * t0) / jnp.maximum(nb, 1.0)
            pool_v[gb >> 3, pl.ds((gb & 7) << 4, D)] = pooled
            return 0

        if h == 0:
            pltpu.sync_copy(
                sqi_hbm.at[pl.ds(wid * 2 * SQ_HALF_CHUNKS + SQ_HALF_CHUNKS,
                                 SQ_HALF_CHUNKS)], sqi_v)
        lax.fori_loop(0, SQ_HALF_B, pool_one, 0)
        if h == 0:
            seq_descs = []
            for j in range(SQ_HALF_CHUNKS):
                seq_descs.append(pltpu.async_copy(
                    stab_hbm.at[sqi_v.at[j]],
                    seq_v.at[pl.ds(j * CH, CH)], sem2))

    pltpu.sync_copy(pool_v, out_pool_hbm.at[pl.ds(wid * (BW * D // 128),
                                                  BW * D // 128)])


def _sc_gather(tab128, seq_table, enc, sqi, sqp):
    mesh = plsc.VectorSubcoreMesh(core_axis_name="c", subcore_axis_name="s")
    f = pl.kernel(
        _sc_body,
        out_type=[
            jax.ShapeDtypeStruct((NW * 2 * STG_ROWS, 128), jnp.float32),
            jax.ShapeDtypeStruct((B * D // 128, 128), jnp.float32),
        ],
        mesh=mesh,
        compiler_params=pltpu.CompilerParams(use_tc_tiling_on_sc=False),
        scratch_types=[
            pltpu.VMEM((SP_CHUNKS, CH), jnp.int32),
            pltpu.VMEM((SQ_HALF_CHUNKS, CH), jnp.int32),
            pltpu.VMEM((BW // 2, 2 * LPAD), jnp.int32),
            pltpu.VMEM((SQ_HALF_ROWS, D), jnp.float32),
            pltpu.VMEM((CH, 128), jnp.float32),
            pltpu.VMEM((STG_ROWS, 128), jnp.float32),
            pltpu.VMEM((BW * D // 128, 128), jnp.float32),
            pltpu.VMEM((1, D), jnp.float32),
            pltpu.VMEM((1, CH), jnp.int32),
            pltpu.SemaphoreType.DMA,
            pltpu.SemaphoreType.DMA,
        ],
    )
    return f(tab128, seq_table, enc, sqi, sqp)


def kernel(sparse_idx, seq_idx, dense_x, sparse_tables, seq_table):
    t3 = jnp.transpose(sparse_tables, (0, 2, 1))
    tab128 = lax.reshape(t3, (TAB_ROWS, 128), dimensions=(0, 2, 1))

    enc = (sparse_idx.astype(jnp.int32) + (
        jnp.arange(NS, dtype=jnp.int32) * VOCAB)[None, :]).reshape(
            NW * SP_CHUNKS, CH)
    qi = seq_idx.astype(jnp.int32)
    sqi = qi.reshape(NW * 2 * SQ_HALF_CHUNKS, CH)
    sqp = jnp.pad(qi, ((0, 0), (0, LPAD - L))).reshape(B // 2, 2 * LPAD)

    out_sp, out_pool = _sc_gather(tab128, seq_table, enc, sqi, sqp)
    return jnp.concatenate(
        [out_sp.reshape(B, NS * D), out_pool.reshape(B, D),
         dense_x.astype(jnp.float32)], axis=1)


# speedup vs baseline: 2.8726x; 2.8726x over previous
import jax
import jax.numpy as jnp
from jax import lax
from jax.experimental import pallas as pl
from jax.experimental.pallas import tpu as pltpu
from jax.experimental.pallas import tpu_sc as plsc

B = 4096
NS = 26
VOCAB = 100000
D = 16
L = 50
ND = 13

NC = 2
NSUB = 16
NW = NC * NSUB
BW = B // NW
CH = 128
NPLANES = NS * D
PPW = NPLANES // NW
BCH = B // CH
SQ_HALF_B = BW // 2
SQ_HALF_ROWS = SQ_HALF_B * L
SQ_HALF_CHUNKS = SQ_HALF_ROWS // CH
LPAD = 64


def _sc_body(tab_hbm, stab_hbm, sidx_hbm, sqi_hbm, sqp_hbm,
             out_t_hbm, out_pool_hbm,
             idx_a, idx_b, pbuf_a, pbuf_b, sqi_v, sqp_v, seq_v, pool_v,
             t0_v, sem, sem2):
    c = lax.axis_index("c")
    s = lax.axis_index("s")
    wid = s * NC + c

    pltpu.sync_copy(sqp_hbm.at[pl.ds(wid * (BW // 2), BW // 2)], sqp_v)
    pltpu.sync_copy(stab_hbm.at[pl.ds(0, 1)], t0_v)
    t0 = t0_v[0, :]

    pltpu.sync_copy(sqi_hbm.at[pl.ds(wid * 2 * SQ_HALF_CHUNKS,
                                     SQ_HALF_CHUNKS)], sqi_v)
    seq_descs = []
    for j in range(SQ_HALF_CHUNKS):
        seq_descs.append(pltpu.async_copy(
            stab_hbm.at[sqi_v.at[j]],
            seq_v.at[pl.ds(j * CH, CH)], sem2))

    bufs = [(idx_a, pbuf_a), (idx_b, pbuf_b)]
    pend = {}
    for i in range(PPW):
        sl = i % 2
        ib, pb = bufs[sl]
        if sl in pend:
            descs, pprev = pend.pop(sl)
            for d_ in descs:
                d_.wait()
            pltpu.sync_copy(pb, out_t_hbm.at[pl.ds(pprev * BCH, BCH)])
        p = wid * PPW + i
        pltpu.sync_copy(sidx_hbm.at[pl.ds(p * BCH, BCH)], ib)
        pend[sl] = ([pltpu.async_copy(tab_hbm.at[ib.at[ch]], pb.at[ch], sem)
                     for ch in range(BCH)], p)
    for sl in (PPW % 2, (PPW + 1) % 2):
        if sl in pend:
            descs, pprev = pend.pop(sl)
            for d_ in descs:
                d_.wait()
            pltpu.sync_copy(bufs[sl][1],
                            out_t_hbm.at[pl.ds(pprev * BCH, BCH)])

    for h in range(2):
        for d_ in seq_descs:
            d_.wait()

        def pool_one(b, _):
            acc = seq_v[b * L, :]
            for l in range(1, L):
                acc = acc + seq_v[b * L + l, :]
            gb = h * SQ_HALF_B + b
            nvec = jnp.zeros((D,), jnp.int32)
            for ch in range(LPAD // D):
                ids = sqp_v[gb // 2, pl.ds((gb % 2) * LPAD + ch * D, D)]
                nvec = nvec + jnp.where(ids > 0, 1, 0).astype(jnp.int32)
            n = nvec[0]
            for i in range(1, D):
                n = n + nvec[i]
            nb = lax.broadcast_in_dim(n.astype(jnp.float32), (D,), ())
            pooled = (acc - (50.0 - nb) * t0) / jnp.maximum(nb, 1.0)
            pool_v[gb >> 3, pl.ds((gb & 7) << 4, D)] = pooled
            return 0

        if h == 0:
            pltpu.sync_copy(
                sqi_hbm.at[pl.ds(wid * 2 * SQ_HALF_CHUNKS + SQ_HALF_CHUNKS,
                                 SQ_HALF_CHUNKS)], sqi_v)
        lax.fori_loop(0, SQ_HALF_B, pool_one, 0)
        if h == 0:
            seq_descs = []
            for j in range(SQ_HALF_CHUNKS):
                seq_descs.append(pltpu.async_copy(
                    stab_hbm.at[sqi_v.at[j]],
                    seq_v.at[pl.ds(j * CH, CH)], sem2))

    pltpu.sync_copy(pool_v, out_pool_hbm.at[pl.ds(wid * (BW * D // 128),
                                                  BW * D // 128)])


def _sc_gather(tab1d, seq_table, sidx, sqi, sqp):
    mesh = plsc.VectorSubcoreMesh(core_axis_name="c", subcore_axis_name="s")
    f = pl.kernel(
        _sc_body,
        out_type=[
            jax.ShapeDtypeStruct((NPLANES * BCH, 128), jnp.float32),
            jax.ShapeDtypeStruct((B * D // 128, 128), jnp.float32),
        ],
        mesh=mesh,
        compiler_params=pltpu.CompilerParams(use_tc_tiling_on_sc=False),
        scratch_types=[
            pltpu.VMEM((BCH, CH), jnp.int32),
            pltpu.VMEM((BCH, CH), jnp.int32),
            pltpu.VMEM((BCH, CH), jnp.float32),
            pltpu.VMEM((BCH, CH), jnp.float32),
            pltpu.VMEM((SQ_HALF_CHUNKS, CH), jnp.int32),
            pltpu.VMEM((BW // 2, 2 * LPAD), jnp.int32),
            pltpu.VMEM((SQ_HALF_ROWS, D), jnp.float32),
            pltpu.VMEM((BW * D // 128, 128), jnp.float32),
            pltpu.VMEM((1, D), jnp.float32),
            pltpu.SemaphoreType.DMA,
            pltpu.SemaphoreType.DMA,
        ],
    )
    return f(tab1d, seq_table, sidx, sqi, sqp)


def kernel(sparse_idx, seq_idx, dense_x, sparse_tables, seq_table):
    tab1d = jnp.transpose(sparse_tables, (0, 2, 1)).reshape(-1)

    sidx_t = sparse_idx.astype(jnp.int32).T
    offs = (jnp.arange(NS, dtype=jnp.int32)[:, None] * D
            + jnp.arange(D, dtype=jnp.int32)[None, :]) * VOCAB
    sidx_all = (sidx_t[:, None, :] + offs[:, :, None]).reshape(
        NPLANES * BCH, CH)

    qi = seq_idx.astype(jnp.int32)
    sqi = qi.reshape(NW * 2 * SQ_HALF_CHUNKS, CH)
    sqp = jnp.pad(qi, ((0, 0), (0, LPAD - L))).reshape(B // 2, 2 * LPAD)

    out_t, out_pool = _sc_gather(tab1d, seq_table, sidx_all, sqi, sqp)
    sp = out_t.reshape(NS, D, B).transpose(2, 0, 1).reshape(B, NS * D)
    return jnp.concatenate(
        [sp, out_pool.reshape(B, D), dense_x.astype(jnp.float32)], axis=1)


# speedup vs baseline: 2.9977x; 1.0436x over previous
import jax
import jax.numpy as jnp
from jax import lax
from jax.experimental import pallas as pl
from jax.experimental.pallas import tpu as pltpu
from jax.experimental.pallas import tpu_sc as plsc

B = 4096
NS = 26
VOCAB = 100000
D = 16
L = 50
ND = 13

NC = 2
NSUB = 16
NW = NC * NSUB
BW = B // NW
CH = 128
NPLANES = NS * D
PPW = NPLANES // NW
BCH = B // CH
SQ_ROWS = BW * L
SQ_CHUNKS = SQ_ROWS // CH
LPAD = 64


def _seq_body(stab_hbm, sqi_hbm, sqp_hbm, out_pool_hbm,
              sqi_v, sqp_v, seq_v, pool_v, t0_v, sem):
    c = lax.axis_index("c")
    s = lax.axis_index("s")
    wid = s * NC + c

    pltpu.sync_copy(sqi_hbm.at[pl.ds(wid * SQ_CHUNKS, SQ_CHUNKS)], sqi_v)
    pltpu.sync_copy(sqp_hbm.at[pl.ds(wid * (BW // 2), BW // 2)], sqp_v)
    pltpu.sync_copy(stab_hbm.at[pl.ds(0, 1)], t0_v)
    t0 = t0_v[0, :]

    descs = []
    for j in range(SQ_CHUNKS):
        descs.append(pltpu.async_copy(
            stab_hbm.at[sqi_v.at[j]],
            seq_v.at[pl.ds(j * CH, CH)], sem))
    for d_ in descs:
        d_.wait()

    def pool_one(b, _):
        acc = seq_v[b * L, :]
        for l in range(1, L):
            acc = acc + seq_v[b * L + l, :]
        nvec = jnp.zeros((D,), jnp.int32)
        for ch in range(LPAD // D):
            ids = sqp_v[b // 2, pl.ds((b % 2) * LPAD + ch * D, D)]
            nvec = nvec + jnp.where(ids > 0, 1, 0).astype(jnp.int32)
        n = nvec[0]
        for i in range(1, D):
            n = n + nvec[i]
        nb = lax.broadcast_in_dim(n.astype(jnp.float32), (D,), ())
        pooled = (acc - (50.0 - nb) * t0) / jnp.maximum(nb, 1.0)
        pool_v[b >> 3, pl.ds((b & 7) << 4, D)] = pooled
        return 0

    lax.fori_loop(0, BW, pool_one, 0)
    pltpu.sync_copy(pool_v, out_pool_hbm.at[pl.ds(wid * (BW * D // 128),
                                                  BW * D // 128)])


def _sp_body(tab_hbm, sidx_hbm, out_t_hbm,
             idx_a, idx_b, pbuf_a, pbuf_b, sem):
    c = lax.axis_index("c")
    s = lax.axis_index("s")
    wid = s * NC + c

    bufs = [(idx_a, pbuf_a), (idx_b, pbuf_b)]
    pend = {}
    for i in range(PPW):
        sl = i % 2
        ib, pb = bufs[sl]
        if sl in pend:
            descs, pprev = pend.pop(sl)
            for d_ in descs:
                d_.wait()
            pltpu.sync_copy(pb, out_t_hbm.at[pl.ds(pprev * BCH, BCH)])
        p = wid * PPW + i
        pltpu.sync_copy(sidx_hbm.at[pl.ds(p * BCH, BCH)], ib)
        pend[sl] = ([pltpu.async_copy(tab_hbm.at[ib.at[ch]], pb.at[ch], sem)
                     for ch in range(BCH)], p)
    for sl in (PPW % 2, (PPW + 1) % 2):
        if sl in pend:
            descs, pprev = pend.pop(sl)
            for d_ in descs:
                d_.wait()
            pltpu.sync_copy(bufs[sl][1],
                            out_t_hbm.at[pl.ds(pprev * BCH, BCH)])


def _seq_kernel(seq_table, sqi, sqp):
    mesh = plsc.VectorSubcoreMesh(core_axis_name="c", subcore_axis_name="s")
    f = pl.kernel(
        _seq_body,
        out_type=[jax.ShapeDtypeStruct((B * D // 128, 128), jnp.float32)],
        mesh=mesh,
        compiler_params=pltpu.CompilerParams(use_tc_tiling_on_sc=False),
        scratch_types=[
            pltpu.VMEM((SQ_CHUNKS, CH), jnp.int32),
            pltpu.VMEM((BW // 2, 2 * LPAD), jnp.int32),
            pltpu.VMEM((SQ_ROWS, D), jnp.float32),
            pltpu.VMEM((BW * D // 128, 128), jnp.float32),
            pltpu.VMEM((1, D), jnp.float32),
            pltpu.SemaphoreType.DMA,
        ],
    )
    return f(seq_table, sqi, sqp)


def _sp_kernel(tab1d, sidx):
    mesh = plsc.VectorSubcoreMesh(core_axis_name="c", subcore_axis_name="s")
    f = pl.kernel(
        _sp_body,
        out_type=[jax.ShapeDtypeStruct((NPLANES * BCH, 128), jnp.float32)],
        mesh=mesh,
        compiler_params=pltpu.CompilerParams(use_tc_tiling_on_sc=False),
        scratch_types=[
            pltpu.VMEM((BCH, CH), jnp.int32),
            pltpu.VMEM((BCH, CH), jnp.int32),
            pltpu.VMEM((BCH, CH), jnp.float32),
            pltpu.VMEM((BCH, CH), jnp.float32),
            pltpu.SemaphoreType.DMA,
        ],
    )
    return f(tab1d, sidx)


def kernel(sparse_idx, seq_idx, dense_x, sparse_tables, seq_table):
    qi = seq_idx.astype(jnp.int32)
    sqi = qi.reshape(NW * SQ_CHUNKS, CH)
    sqp = jnp.pad(qi, ((0, 0), (0, LPAD - L))).reshape(B // 2, 2 * LPAD)
    (out_pool,) = _seq_kernel(seq_table, sqi, sqp)

    tab1d = jnp.transpose(sparse_tables, (0, 2, 1)).reshape(-1)
    sidx_t = sparse_idx.astype(jnp.int32).T
    offs = (jnp.arange(NS, dtype=jnp.int32)[:, None] * D
            + jnp.arange(D, dtype=jnp.int32)[None, :]) * VOCAB
    sidx_all = (sidx_t[:, None, :] + offs[:, :, None]).reshape(
        NPLANES * BCH, CH)
    (out_t,) = _sp_kernel(tab1d, sidx_all)

    sp = out_t.reshape(NS, D, B).transpose(2, 0, 1).reshape(B, NS * D)
    return jnp.concatenate(
        [sp, out_pool.reshape(B, D), dense_x.astype(jnp.float32)], axis=1)


# speedup vs baseline: 3.0182x; 1.0068x over previous
import jax
import jax.numpy as jnp
from jax import lax
from jax.experimental import pallas as pl
from jax.experimental.pallas import tpu as pltpu
from jax.experimental.pallas import tpu_sc as plsc

B = 4096
NS = 26
VOCAB = 100000
D = 16
L = 50
ND = 13

NC = 2
NSUB = 16
NW = NC * NSUB
BW = B // NW
CH = 128
NPLANES = NS * D
PPW = NPLANES // NW
BCH = B // CH
SQ_ROWS = BW * L
SQ_CHUNKS = SQ_ROWS // CH
LPAD = 64


def _seq_body(stab_hbm, sqi_hbm, sqp_hbm, out_pool_hbm,
              sqi_v, sqp_v, seq_v, pool_v, t0_v, sem):
    c = lax.axis_index("c")
    s = lax.axis_index("s")
    wid = s * NC + c

    pltpu.sync_copy(sqi_hbm.at[pl.ds(wid * SQ_CHUNKS, SQ_CHUNKS)], sqi_v)
    pltpu.sync_copy(sqp_hbm.at[pl.ds(wid * (BW // 2), BW // 2)], sqp_v)
    pltpu.sync_copy(stab_hbm.at[pl.ds(0, 1)], t0_v)
    t0 = t0_v[0, :]

    descs = []
    for j in range(SQ_CHUNKS):
        descs.append(pltpu.async_copy(
            stab_hbm.at[sqi_v.at[j]],
            seq_v.at[pl.ds(j * CH, CH)], sem))
    for d_ in descs:
        d_.wait()

    def pool_one(b, _):
        acc = seq_v[b * L, :]
        for l in range(1, L):
            acc = acc + seq_v[b * L + l, :]
        nvec = jnp.zeros((D,), jnp.int32)
        for ch in range(LPAD // D):
            ids = sqp_v[b // 2, pl.ds((b % 2) * LPAD + ch * D, D)]
            nvec = nvec + jnp.where(ids > 0, 1, 0).astype(jnp.int32)
        n = nvec[0]
        for i in range(1, D):
            n = n + nvec[i]
        nb = lax.broadcast_in_dim(n.astype(jnp.float32), (D,), ())
        pooled = (acc - (50.0 - nb) * t0) / jnp.maximum(nb, 1.0)
        pool_v[b >> 3, pl.ds((b & 7) << 4, D)] = pooled
        return 0

    lax.fori_loop(0, BW, pool_one, 0)
    pltpu.sync_copy(pool_v, out_pool_hbm.at[pl.ds(wid * (BW * D // 128),
                                                  BW * D // 128)])


def _sp_body(tab_hbm, sidx_hbm, out_t_hbm,
             idx_a, idx_b, pbuf_a, pbuf_b, sem):
    c = lax.axis_index("c")
    s = lax.axis_index("s")
    wid = s * NC + c

    bufs = [(idx_a, pbuf_a), (idx_b, pbuf_b)]
    pend = {}
    for i in range(PPW):
        sl = i % 2
        ib, pb = bufs[sl]
        if sl in pend:
            descs, pprev = pend.pop(sl)
            for d_ in descs:
                d_.wait()
            pltpu.sync_copy(pb, out_t_hbm.at[pl.ds(pprev * B, B)])
        p = wid * PPW + i
        pltpu.sync_copy(sidx_hbm.at[pl.ds(p * B, B)], ib)
        pend[sl] = ([pltpu.async_copy(tab_hbm.at[ib], pb, sem)], p)
    for sl in (PPW % 2, (PPW + 1) % 2):
        if sl in pend:
            descs, pprev = pend.pop(sl)
            for d_ in descs:
                d_.wait()
            pltpu.sync_copy(bufs[sl][1],
                            out_t_hbm.at[pl.ds(pprev * B, B)])


def _seq_kernel(seq_table, sqi, sqp):
    mesh = plsc.VectorSubcoreMesh(core_axis_name="c", subcore_axis_name="s")
    f = pl.kernel(
        _seq_body,
        out_type=[jax.ShapeDtypeStruct((B * D // 128, 128), jnp.float32)],
        mesh=mesh,
        compiler_params=pltpu.CompilerParams(use_tc_tiling_on_sc=False),
        scratch_types=[
            pltpu.VMEM((SQ_CHUNKS, CH), jnp.int32),
            pltpu.VMEM((BW // 2, 2 * LPAD), jnp.int32),
            pltpu.VMEM((SQ_ROWS, D), jnp.float32),
            pltpu.VMEM((BW * D // 128, 128), jnp.float32),
            pltpu.VMEM((1, D), jnp.float32),
            pltpu.SemaphoreType.DMA,
        ],
    )
    return f(seq_table, sqi, sqp)


def _sp_kernel(tab1d, sidx):
    mesh = plsc.VectorSubcoreMesh(core_axis_name="c", subcore_axis_name="s")
    f = pl.kernel(
        _sp_body,
        out_type=[jax.ShapeDtypeStruct((NPLANES * B,), jnp.float32)],
        mesh=mesh,
        compiler_params=pltpu.CompilerParams(use_tc_tiling_on_sc=False),
        scratch_types=[
            pltpu.VMEM((B,), jnp.int32),
            pltpu.VMEM((B,), jnp.int32),
            pltpu.VMEM((B,), jnp.float32),
            pltpu.VMEM((B,), jnp.float32),
            pltpu.SemaphoreType.DMA,
        ],
    )
    return f(tab1d, sidx)


def kernel(sparse_idx, seq_idx, dense_x, sparse_tables, seq_table):
    qi = seq_idx.astype(jnp.int32)
    sqi = qi.reshape(NW * SQ_CHUNKS, CH)
    sqp = jnp.pad(qi, ((0, 0), (0, LPAD - L))).reshape(B // 2, 2 * LPAD)
    (out_pool,) = _seq_kernel(seq_table, sqi, sqp)

    tab1d = jnp.transpose(sparse_tables, (0, 2, 1)).reshape(-1)
    sidx_t = sparse_idx.astype(jnp.int32).T
    offs = (jnp.arange(NS, dtype=jnp.int32)[:, None] * D
            + jnp.arange(D, dtype=jnp.int32)[None, :]) * VOCAB
    sidx_all = (sidx_t[:, None, :] + offs[:, :, None]).reshape(-1)
    (out_t,) = _sp_kernel(tab1d, sidx_all)

    sp = out_t.reshape(NS, D, B).transpose(2, 0, 1).reshape(B, NS * D)
    return jnp.concatenate(
        [sp, out_pool.reshape(B, D), dense_x.astype(jnp.float32)], axis=1)
